# baseline (speedup 1.0000x reference)
import functools

import jax
import jax.numpy as jnp
import numpy as np
from jax.experimental import pallas as pl

N = 10000
NUM_COEF = 16
C = 32
H = 32
HEADS = 4
VC = 8

M0 = np.array([0, 2, 6, 12])
MP1 = np.array([3, 7, 13])
MN1 = np.array([3 - 2, 7 - 2, 13 - 2])
MP2 = np.array([8, 14])
MN2 = np.array([8 - 4, 14 - 4])


def _out_matmul_body(agg_ref, w_ref, o_ref):
    o_ref[...] = jnp.dot(agg_ref[...], w_ref[...],
                         preferred_element_type=jnp.float32)


def _final_matmul(agg, W_out):
    rows = N * NUM_COEF
    blk = 8000
    a2 = agg.reshape(rows, H)
    out = pl.pallas_call(
        _out_matmul_body,
        grid=(rows // blk,),
        in_specs=[pl.BlockSpec((blk, H), lambda i: (i, 0)),
                  pl.BlockSpec((H, C), lambda i: (0, 0))],
        out_specs=pl.BlockSpec((blk, C), lambda i: (i, 0)),
        out_shape=jax.ShapeDtypeStruct((rows, C), jnp.float32),
    )(a2, W_out)
    return out.reshape(N, NUM_COEF, C)


def kernel(x, edge_index, edge_distance_rbf, wigner, W_rbf, b_rbf, w0, w1r,
           w1i, w2r, w2i, Wa, alpha_dot, W_out):
    src = edge_index[0]
    dst = edge_index[1]
    x_src = jnp.einsum('eij,ejc->eic', wigner, x[src])
    x_dst = jnp.einsum('eij,ejc->eic', wigner, x[dst])
    xe = jnp.concatenate([x_src, x_dst], axis=-1)
    e = xe.shape[0]
    edge_emb = jax.nn.relu(edge_distance_rbf @ W_rbf + b_rbf)
    m0_in = jnp.concatenate([xe[:, M0, :].reshape(e, -1), edge_emb], axis=-1)
    y0 = (m0_in @ w0).reshape(e, 4, H)
    xp1 = xe[:, MP1, :].reshape(e, -1)
    xn1 = xe[:, MN1, :].reshape(e, -1)
    yp1 = (xp1 @ w1r - xn1 @ w1i).reshape(e, 3, H)
    yn1 = (xn1 @ w1r + xp1 @ w1i).reshape(e, 3, H)
    xp2 = xe[:, MP2, :].reshape(e, -1)
    xn2 = xe[:, MN2, :].reshape(e, -1)
    yp2 = (xp2 @ w2r - xn2 @ w2i).reshape(e, 2, H)
    yn2 = (xn2 @ w2r + xp2 @ w2i).reshape(e, 2, H)
    a = jax.nn.silu(y0.reshape(e, -1) @ Wa).reshape(e, HEADS, 16)
    logits = jnp.sum(a * alpha_dot[None], axis=-1)
    amax = jax.ops.segment_max(logits, dst, num_segments=N)
    ex = jnp.exp(logits - amax[dst])
    denom = jax.ops.segment_sum(ex, dst, num_segments=N)[dst] + 1e-9
    attn = ex / denom
    zero_col = jnp.zeros((e, H), dtype=x.dtype)
    msg_cols = [zero_col] * NUM_COEF
    for j, idx in enumerate(M0):
        msg_cols[idx] = y0[:, j, :]
    for j, idx in enumerate(MP1):
        msg_cols[idx] = yp1[:, j, :]
    for j, idx in enumerate(MN1):
        msg_cols[idx] = yn1[:, j, :]
    for j, idx in enumerate(MP2):
        msg_cols[idx] = yp2[:, j, :]
    for j, idx in enumerate(MN2):
        msg_cols[idx] = yn2[:, j, :]
    msg = jnp.stack(msg_cols, axis=1)
    msg = (msg.reshape(e, NUM_COEF, HEADS, VC) * attn[:, None, :, None]
           ).reshape(e, NUM_COEF, H)
    msg = jnp.einsum('eji,ejc->eic', wigner, msg)
    agg = jax.ops.segment_sum(msg, dst, num_segments=N)
    return _final_matmul(agg, W_out)


# speedup vs baseline: 1.0112x; 1.0112x over previous
import functools

import jax
import jax.numpy as jnp
import numpy as np
from jax import lax
from jax.experimental import pallas as pl
from jax.experimental.pallas import tpu as pltpu
from jax.experimental.pallas import tpu_sc as plsc

N = 10000
E = 100000
NUM_COEF = 16
C = 32
H = 32
HEADS = 4
VC = 8
D = NUM_COEF * C

NC = 2
NS = 16
NW = NC * NS

EP = 100352
BPW = EP // NW
GCH = 112

M0 = np.array([0, 2, 6, 12])
MP1 = np.array([3, 7, 13])
MN1 = np.array([1, 5, 11])
MP2 = np.array([8, 14])
MN2 = np.array([4, 10])

_sc_mesh = functools.partial(
    pl.kernel,
    mesh=plsc.VectorSubcoreMesh(core_axis_name="c", subcore_axis_name="s"),
)


@functools.partial(
    _sc_mesh,
    out_type=(jax.ShapeDtypeStruct((EP, D), jnp.float32),
              jax.ShapeDtypeStruct((EP, D), jnp.float32)),
    scratch_types=[pltpu.VMEM((GCH,), jnp.int32),
                   pltpu.VMEM((GCH, D), jnp.float32),
                   pltpu.VMEM((GCH,), jnp.int32),
                   pltpu.VMEM((GCH, D), jnp.float32),
                   pltpu.SemaphoreType.DMA,
                   pltpu.SemaphoreType.DMA],
)
def _sc_gather(table_hbm, src_hbm, dst_hbm, outs_hbm, outd_hbm,
               idxs_v, rows_s, idxd_v, rows_d, sems, semd):
    wid = lax.axis_index("s") * NC + lax.axis_index("c")
    wbase = wid * BPW

    def body(k, _):
        base = wbase + k * GCH
        pltpu.sync_copy(src_hbm.at[pl.ds(base, GCH)], idxs_v)
        pltpu.sync_copy(dst_hbm.at[pl.ds(base, GCH)], idxd_v)
        cs = pltpu.async_copy(table_hbm.at[idxs_v], rows_s, sems)
        cd = pltpu.async_copy(table_hbm.at[idxd_v], rows_d, semd)
        cs.wait()
        pltpu.sync_copy(rows_s, outs_hbm.at[pl.ds(base, GCH)])
        cd.wait()
        pltpu.sync_copy(rows_d, outd_hbm.at[pl.ds(base, GCH)])
        return ()

    lax.fori_loop(0, BPW // GCH, body, ())


def _out_matmul_body(agg_ref, w_ref, o_ref):
    o_ref[...] = jnp.dot(agg_ref[...], w_ref[...],
                         preferred_element_type=jnp.float32)


def _final_matmul(agg, W_out):
    rows = N * NUM_COEF
    blk = 8000
    a2 = agg.reshape(rows, H)
    out = pl.pallas_call(
        _out_matmul_body,
        grid=(rows // blk,),
        in_specs=[pl.BlockSpec((blk, H), lambda i: (i, 0)),
                  pl.BlockSpec((H, C), lambda i: (0, 0))],
        out_specs=pl.BlockSpec((blk, C), lambda i: (i, 0)),
        out_shape=jax.ShapeDtypeStruct((rows, C), jnp.float32),
    )(a2, W_out)
    return out.reshape(N, NUM_COEF, C)


def kernel(x, edge_index, edge_distance_rbf, wigner, W_rbf, b_rbf, w0, w1r,
           w1i, w2r, w2i, Wa, alpha_dot, W_out):
    dst = edge_index[1]
    table = x.reshape(N, D)
    src_pad = jnp.pad(edge_index[0], (0, EP - E))
    dst_pad = jnp.pad(edge_index[1], (0, EP - E))
    xs_flat, xd_flat = _sc_gather(table, src_pad, dst_pad)
    x_src = jnp.einsum('eij,ejc->eic', wigner,
                       xs_flat[:E].reshape(E, NUM_COEF, C))
    x_dst = jnp.einsum('eij,ejc->eic', wigner,
                       xd_flat[:E].reshape(E, NUM_COEF, C))
    xe = jnp.concatenate([x_src, x_dst], axis=-1)
    e = xe.shape[0]
    edge_emb = jax.nn.relu(edge_distance_rbf @ W_rbf + b_rbf)
    m0_in = jnp.concatenate([xe[:, M0, :].reshape(e, -1), edge_emb], axis=-1)
    y0 = (m0_in @ w0).reshape(e, 4, H)
    xp1 = xe[:, MP1, :].reshape(e, -1)
    xn1 = xe[:, MN1, :].reshape(e, -1)
    yp1 = (xp1 @ w1r - xn1 @ w1i).reshape(e, 3, H)
    yn1 = (xn1 @ w1r + xp1 @ w1i).reshape(e, 3, H)
    xp2 = xe[:, MP2, :].reshape(e, -1)
    xn2 = xe[:, MN2, :].reshape(e, -1)
    yp2 = (xp2 @ w2r - xn2 @ w2i).reshape(e, 2, H)
    yn2 = (xn2 @ w2r + xp2 @ w2i).reshape(e, 2, H)
    a = jax.nn.silu(y0.reshape(e, -1) @ Wa).reshape(e, HEADS, 16)
    logits = jnp.sum(a * alpha_dot[None], axis=-1)
    amax = jax.ops.segment_max(logits, dst, num_segments=N)
    ex = jnp.exp(logits - amax[dst])
    denom = jax.ops.segment_sum(ex, dst, num_segments=N)[dst] + 1e-9
    attn = ex / denom
    zero_col = jnp.zeros((e, H), dtype=x.dtype)
    msg_cols = [zero_col] * NUM_COEF
    for j, idx in enumerate(M0):
        msg_cols[idx] = y0[:, j, :]
    for j, idx in enumerate(MP1):
        msg_cols[idx] = yp1[:, j, :]
    for j, idx in enumerate(MN1):
        msg_cols[idx] = yn1[:, j, :]
    for j, idx in enumerate(MP2):
        msg_cols[idx] = yp2[:, j, :]
    for j, idx in enumerate(MN2):
        msg_cols[idx] = yn2[:, j, :]
    msg = jnp.stack(msg_cols, axis=1)
    msg = (msg.reshape(e, NUM_COEF, HEADS, VC) * attn[:, None, :, None]
           ).reshape(e, NUM_COEF, H)
    msg = jnp.einsum('eji,ejc->eic', wigner, msg)
    agg = jax.ops.segment_sum(msg, dst, num_segments=N)
    return _final_matmul(agg, W_out)


# speedup vs baseline: 6.4060x; 6.3349x over previous
import functools

import jax
import jax.numpy as jnp
import numpy as np
from jax import lax
from jax.experimental import pallas as pl
from jax.experimental.pallas import tpu as pltpu
from jax.experimental.pallas import tpu_sc as plsc

N = 10000
E = 100000
NUM_COEF = 16
C = 32
H = 32
HEADS = 4
VC = 8
D = NUM_COEF * C

NC = 2
NS = 16
NW = NC * NS

EP = 100352
BPW = EP // NW
GCH = 112

M0 = np.array([0, 2, 6, 12])
MP1 = np.array([3, 7, 13])
MN1 = np.array([1, 5, 11])
MP2 = np.array([8, 14])
MN2 = np.array([4, 10])

_sc_mesh = functools.partial(
    pl.kernel,
    mesh=plsc.VectorSubcoreMesh(core_axis_name="c", subcore_axis_name="s"),
)


@functools.partial(
    _sc_mesh,
    out_type=(jax.ShapeDtypeStruct((EP, D), jnp.float32),
              jax.ShapeDtypeStruct((EP, D), jnp.float32)),
    scratch_types=[pltpu.VMEM((GCH,), jnp.int32),
                   pltpu.VMEM((GCH, D), jnp.float32),
                   pltpu.VMEM((GCH,), jnp.int32),
                   pltpu.VMEM((GCH, D), jnp.float32),
                   pltpu.SemaphoreType.DMA,
                   pltpu.SemaphoreType.DMA],
)
def _sc_gather(table_hbm, src_hbm, dst_hbm, outs_hbm, outd_hbm,
               idxs_v, rows_s, idxd_v, rows_d, sems, semd):
    wid = lax.axis_index("s") * NC + lax.axis_index("c")
    wbase = wid * BPW

    def body(k, _):
        base = wbase + k * GCH
        pltpu.sync_copy(src_hbm.at[pl.ds(base, GCH)], idxs_v)
        pltpu.sync_copy(dst_hbm.at[pl.ds(base, GCH)], idxd_v)
        cs = pltpu.async_copy(table_hbm.at[idxs_v], rows_s, sems)
        cd = pltpu.async_copy(table_hbm.at[idxd_v], rows_d, semd)
        cs.wait()
        pltpu.sync_copy(rows_s, outs_hbm.at[pl.ds(base, GCH)])
        cd.wait()
        pltpu.sync_copy(rows_d, outd_hbm.at[pl.ds(base, GCH)])
        return ()

    lax.fori_loop(0, BPW // GCH, body, ())


BE = 128
NB = EP // BE
OUTW = 640
COEFS = [0, 1, 2, 3, 4, 5, 6, 7, 8, 10, 11, 12, 13, 14]


def _edge_body(xs_ref, xd_ref, wg_ref, rbf_ref, w0_ref, w1r_ref, w1i_ref,
               w2r_ref, w2i_ref, wa_ref, wrbf_ref, brbf_ref, ad_ref, o_ref):
    xsT = xs_ref[...].T
    xdT = xd_ref[...].T
    wgT = wg_ref[...].T
    rbT = rbf_ref[...].T

    embT = jnp.maximum(
        jnp.dot(wrbf_ref[...], rbT, preferred_element_type=jnp.float32)
        + brbf_ref[...], 0.0)

    rot = {}
    for i in COEFS:
        acc_s = None
        acc_d = None
        for j in range(16):
            w_ij = wgT[i * 16 + j:i * 16 + j + 1, :]
            ts = w_ij * xsT[32 * j:32 * j + 32, :]
            td = w_ij * xdT[32 * j:32 * j + 32, :]
            acc_s = ts if acc_s is None else acc_s + ts
            acc_d = td if acc_d is None else acc_d + td
        rot[i] = (acc_s, acc_d)

    m0_inT = jnp.concatenate(
        [blk for k in M0 for blk in rot[int(k)]] + [embT], axis=0)
    y0T = jnp.dot(w0_ref[...], m0_inT, preferred_element_type=jnp.float32)

    xp1T = jnp.concatenate([blk for k in MP1 for blk in rot[int(k)]], axis=0)
    xn1T = jnp.concatenate([blk for k in MN1 for blk in rot[int(k)]], axis=0)
    yp1T = (jnp.dot(w1r_ref[...], xp1T, preferred_element_type=jnp.float32)
            - jnp.dot(w1i_ref[...], xn1T, preferred_element_type=jnp.float32))
    yn1T = (jnp.dot(w1r_ref[...], xn1T, preferred_element_type=jnp.float32)
            + jnp.dot(w1i_ref[...], xp1T, preferred_element_type=jnp.float32))
    xp2T = jnp.concatenate([blk for k in MP2 for blk in rot[int(k)]], axis=0)
    xn2T = jnp.concatenate([blk for k in MN2 for blk in rot[int(k)]], axis=0)
    yp2T = (jnp.dot(w2r_ref[...], xp2T, preferred_element_type=jnp.float32)
            - jnp.dot(w2i_ref[...], xn2T, preferred_element_type=jnp.float32))
    yn2T = (jnp.dot(w2r_ref[...], xn2T, preferred_element_type=jnp.float32)
            + jnp.dot(w2i_ref[...], xp2T, preferred_element_type=jnp.float32))

    zT = jnp.dot(wa_ref[...], y0T, preferred_element_type=jnp.float32)
    aT = zT / (1.0 + jnp.exp(-zT)) * ad_ref[...]
    heads = []
    for h in range(HEADS):
        u = aT[16 * h:16 * h + 8, :] + aT[16 * h + 8:16 * h + 16, :]
        u = u[0:4, :] + u[4:8, :]
        u = u[0:2, :] + u[2:4, :]
        heads.append(u[0:1, :] + u[1:2, :])
    logitsT = jnp.concatenate(heads, axis=0)

    eids = (pl.program_id(0) * BE
            + jax.lax.broadcasted_iota(jnp.int32, (1, BE), 1))
    exT = jnp.where(eids < E, jnp.exp(logitsT), 0.0)
    exrepT = jnp.concatenate([exT[h:h + 1, :] for h in range(HEADS)
                              for _ in range(VC)], axis=0)

    msg = {}
    for j, k in enumerate(M0):
        msg[int(k)] = y0T[32 * j:32 * j + 32, :] * exrepT
    for j, k in enumerate(MP1):
        msg[int(k)] = yp1T[32 * j:32 * j + 32, :] * exrepT
    for j, k in enumerate(MN1):
        msg[int(k)] = yn1T[32 * j:32 * j + 32, :] * exrepT
    for j, k in enumerate(MP2):
        msg[int(k)] = yp2T[32 * j:32 * j + 32, :] * exrepT
    for j, k in enumerate(MN2):
        msg[int(k)] = yn2T[32 * j:32 * j + 32, :] * exrepT

    outs = []
    for i in range(NUM_COEF):
        acc = None
        for j in COEFS:
            t = wgT[j * 16 + i:j * 16 + i + 1, :] * msg[j]
            acc = t if acc is None else acc + t
        outs.append(acc)
    outs.append(exrepT)
    outs.append(jnp.zeros((OUTW - 544, BE), jnp.float32))
    o_ref[...] = jnp.concatenate(outs, axis=0).T


def _edge_kernel(xs, xd, wg2, rbf_pad, w0, w1r, w1i, w2r, w2i, Wa, W_rbf,
                 b_rbf, alpha_dot):
    full = lambda shape: pl.BlockSpec(shape, lambda i: (0, 0))
    return pl.pallas_call(
        _edge_body,
        grid=(NB,),
        in_specs=[pl.BlockSpec((BE, D), lambda i: (i, 0)),
                  pl.BlockSpec((BE, D), lambda i: (i, 0)),
                  pl.BlockSpec((BE, 256), lambda i: (i, 0)),
                  pl.BlockSpec((BE, 64), lambda i: (i, 0)),
                  full((4 * H, 288)), full((3 * H, 192)), full((3 * H, 192)),
                  full((2 * H, 128)), full((2 * H, 128)), full((64, 128)),
                  full((32, 64)), full((32, 1)), full((64, 1))],
        out_specs=pl.BlockSpec((BE, OUTW), lambda i: (i, 0)),
        out_shape=jax.ShapeDtypeStruct((EP, OUTW), jnp.float32),
    )(xs, xd, wg2, rbf_pad, w0.T, w1r.T, w1i.T, w2r.T, w2i.T, Wa.T, W_rbf.T,
      b_rbf.reshape(32, 1), alpha_dot.reshape(64, 1))


def _out_matmul_body(agg_ref, w_ref, o_ref):
    o_ref[...] = jnp.dot(agg_ref[...], w_ref[...],
                         preferred_element_type=jnp.float32)


def _final_matmul(agg, W_out):
    rows = N * NUM_COEF
    blk = 8000
    a2 = agg.reshape(rows, H)
    out = pl.pallas_call(
        _out_matmul_body,
        grid=(rows // blk,),
        in_specs=[pl.BlockSpec((blk, H), lambda i: (i, 0)),
                  pl.BlockSpec((H, C), lambda i: (0, 0))],
        out_specs=pl.BlockSpec((blk, C), lambda i: (i, 0)),
        out_shape=jax.ShapeDtypeStruct((rows, C), jnp.float32),
    )(a2, W_out)
    return out.reshape(N, NUM_COEF, C)


def kernel(x, edge_index, edge_distance_rbf, wigner, W_rbf, b_rbf, w0, w1r,
           w1i, w2r, w2i, Wa, alpha_dot, W_out):
    table = x.reshape(N, D)
    src_pad = jnp.pad(edge_index[0], (0, EP - E))
    dst_pad = jnp.pad(edge_index[1], (0, EP - E))
    xs_flat, xd_flat = _sc_gather(table, src_pad, dst_pad)
    wg2 = wigner.reshape(E, 256)
    wg2 = jnp.pad(wg2, ((0, EP - E), (0, 0)))
    rbf_pad = jnp.pad(edge_distance_rbf, ((0, EP - E), (0, 0)))
    rows = _edge_kernel(xs_flat, xd_flat, wg2, rbf_pad, w0, w1r, w1i, w2r,
                        w2i, Wa, W_rbf, b_rbf, alpha_dot)
    acc = jax.ops.segment_sum(rows, dst_pad, num_segments=N)
    num = acc[:, :D].reshape(N, NUM_COEF, H)
    den = acc[:, D:D + H].reshape(N, 1, H) + 1e-9
    agg = num / den
    return _final_matmul(agg, W_out)
